# baseline (speedup 1.0000x reference)
import functools

import jax
import jax.numpy as jnp
from jax import lax
from jax.experimental import pallas as pl
from jax.experimental.pallas import tpu as pltpu
from jax.experimental.pallas import tpu_sc as plsc

N = 10000
NPAD = 10240
D = 128
NC, NS = 2, 16
NW = NC * NS
EB = 128
RPS = NPAD // NS

_MESH = plsc.VectorSubcoreMesh(core_axis_name="c", subcore_axis_name="s")


def _make_deg_kernel(e_pad):
  nb = e_pad // (NW * EB)

  @functools.partial(
      pl.kernel,
      out_type=jax.ShapeDtypeStruct((NC, NPAD), jnp.float32),
      mesh=_MESH,
      scratch_types=[
          pltpu.VMEM((EB,), jnp.int32),
          pltpu.VMEM((EB,), jnp.float32),
          pltpu.VMEM((RPS,), jnp.float32),
          pltpu.VMEM_SHARED((NPAD,), jnp.float32),
      ],
  )
  def deg_kernel(dst_hbm, out_hbm, idx_v, ones_v, zbuf_v, acc):
    c = lax.axis_index("c")
    s = lax.axis_index("s")
    wid = s * NC + c

    def fill_ones(i, _):
      ones_v[pl.ds(i * 16, 16)] = jnp.ones((16,), jnp.float32)
      return 0

    lax.fori_loop(0, EB // 16, fill_ones, 0)

    def fill_zero(i, _):
      zbuf_v[pl.ds(i * 16, 16)] = jnp.zeros((16,), jnp.float32)
      return 0

    lax.fori_loop(0, RPS // 16, fill_zero, 0)
    pltpu.sync_copy(zbuf_v, acc.at[pl.ds(s * RPS, RPS)])
    plsc.subcore_barrier()

    base = wid * (nb * EB)

    def body(j, _):
      pltpu.sync_copy(dst_hbm.at[pl.ds(base + j * EB, EB)], idx_v)
      pltpu.sync_copy(ones_v, acc.at[idx_v], add=True)
      return 0

    lax.fori_loop(0, nb, body, 0)
    plsc.subcore_barrier()
    pltpu.sync_copy(acc.at[pl.ds(s * RPS, RPS)],
                    out_hbm.at[c, pl.ds(s * RPS, RPS)])

  return deg_kernel


CH = 16


def _make_scatter_kernel(e_pad):
  nb = e_pad // (NW * EB)
  assert nb % CH == 0

  @functools.partial(
      pl.kernel,
      out_type=jax.ShapeDtypeStruct((NC, NPAD, D), jnp.float32),
      mesh=_MESH,
      scratch_types=[
          pltpu.VMEM((CH, EB), jnp.int32),
          pltpu.VMEM((EB,), jnp.int32),
          pltpu.VMEM((EB, D), jnp.float32),
          pltpu.VMEM((EB, D), jnp.float32),
          pltpu.VMEM_SHARED((NPAD, D), jnp.float32),
          pltpu.SemaphoreType.DMA,
          pltpu.SemaphoreType.DMA,
      ],
  )
  def scat_kernel(y_hbm, src_hbm, dst_hbm, out_hbm,
                  sidx_v, didx_v, rows0_v, rows1_v, acc, sem0, sem1):
    c = lax.axis_index("c")
    s = lax.axis_index("s")
    wid = s * NC + c

    def zero_rows(i, _):
      rows0_v[i // 8, pl.ds((i % 8) * 16, 16)] = jnp.zeros((16,), jnp.float32)
      return 0

    lax.fori_loop(0, EB * (D // 16), zero_rows, 0)

    def zero_acc(j, _):
      pltpu.sync_copy(rows0_v, acc.at[pl.ds(s * RPS + j * EB, EB)])
      return 0

    lax.fori_loop(0, RPS // EB, zero_acc, 0)
    plsc.subcore_barrier()

    rows = (rows0_v, rows1_v)
    sems = (sem0, sem1)

    base = wid * (nb * EB)

    def chunk_body(ci, _):
      pltpu.sync_copy(src_hbm.at[wid, pl.ds(ci * CH, CH)], sidx_v)
      pltpu.async_copy(y_hbm.at[sidx_v.at[0]], rows0_v, sem0)
      for k in range(CH):
        b = k % 2
        if k + 1 < CH:
          pltpu.async_copy(y_hbm.at[sidx_v.at[k + 1]], rows[1 - b],
                           sems[1 - b])
        pltpu.sync_copy(
            dst_hbm.at[pl.ds(base + (ci * CH + k) * EB, EB)], didx_v)
        pltpu.make_async_copy(y_hbm.at[sidx_v.at[k]], rows[b], sems[b]).wait()
        pltpu.sync_copy(rows[b], acc.at[didx_v], add=True)
      return 0

    lax.fori_loop(0, nb // CH, chunk_body, 0)
    plsc.subcore_barrier()

    def writeback(j, _):
      pltpu.sync_copy(acc.at[pl.ds(s * RPS + j * EB, EB)],
                      out_hbm.at[c, pl.ds(s * RPS + j * EB, EB)])
      return 0

    lax.fori_loop(0, RPS // EB, writeback, 0)

  return scat_kernel


RB = 1024
_TC_GRID = NPAD // RB


def _lin1_body(x_ref, w_ref, d0_ref, d1_ref, y_ref, dinv_ref):
  deg = d0_ref[...] + d1_ref[...] + 1.0
  dinv = lax.rsqrt(deg)
  dinv_ref[...] = dinv
  xw = jnp.dot(x_ref[...], w_ref[...], preferred_element_type=jnp.float32)
  y_ref[...] = xw * dinv


def _lin2_body(z0_ref, z1_ref, y1_ref, dinv_ref, b1_ref, w_ref, y2_ref):
  dinv = dinv_ref[...]
  z = z0_ref[...] + z1_ref[...] + y1_ref[...]
  h = jnp.maximum(z * dinv + b1_ref[...], 0.0)
  hw = jnp.dot(h, w_ref[...], preferred_element_type=jnp.float32)
  y2_ref[...] = hw * dinv


def _out_body(z0_ref, z1_ref, y2_ref, dinv_ref, b2_ref, o_ref):
  z = z0_ref[...] + z1_ref[...] + y2_ref[...]
  o_ref[...] = z * dinv_ref[...] + b2_ref[...]


def _rows_spec():
  return pl.BlockSpec((RB, D), lambda i: (i, 0))


def _col_spec():
  return pl.BlockSpec((RB, 1), lambda i: (i, 0))


def _full_spec(r):
  return pl.BlockSpec((r, D), lambda i: (0, 0))


def kernel(x, edge_index, W1, b1, W2, b2):
  src = edge_index[0].astype(jnp.int32)
  dst = edge_index[1].astype(jnp.int32)
  e = src.shape[0]
  chunk = CH * EB * NW
  e_pad = chunk * (-(-e // chunk))
  nb = e_pad // (NW * EB)
  pad = N + (jnp.arange(e_pad - e, dtype=jnp.int32) % (NPAD - N))
  src_p = jnp.concatenate([src, pad])
  dst_p = jnp.concatenate([dst, pad])
  src3 = src_p.reshape(NW, nb, EB)
  x_p = jnp.pad(x, ((0, NPAD - N), (0, 0)))

  deg_fn = _make_deg_kernel(e_pad)
  scat_fn = _make_scatter_kernel(e_pad)

  degp = deg_fn(dst_p)
  d0 = degp[0][:, None]
  d1 = degp[1][:, None]

  y1, dinv = pl.pallas_call(
      _lin1_body,
      grid=(_TC_GRID,),
      in_specs=[_rows_spec(), _full_spec(D), _col_spec(), _col_spec()],
      out_specs=[_rows_spec(), _col_spec()],
      out_shape=[
          jax.ShapeDtypeStruct((NPAD, D), jnp.float32),
          jax.ShapeDtypeStruct((NPAD, 1), jnp.float32),
      ],
  )(x_p, W1, d0, d1)

  zp1 = scat_fn(y1, src3, dst_p)

  y2 = pl.pallas_call(
      _lin2_body,
      grid=(_TC_GRID,),
      in_specs=[_rows_spec(), _rows_spec(), _rows_spec(), _col_spec(),
                _full_spec(1), _full_spec(D)],
      out_specs=_rows_spec(),
      out_shape=jax.ShapeDtypeStruct((NPAD, D), jnp.float32),
  )(zp1[0], zp1[1], y1, dinv, b1.reshape(1, D), W2)

  zp2 = scat_fn(y2, src3, dst_p)

  out = pl.pallas_call(
      _out_body,
      grid=(_TC_GRID,),
      in_specs=[_rows_spec(), _rows_spec(), _rows_spec(), _col_spec(),
                _full_spec(1)],
      out_specs=_rows_spec(),
      out_shape=jax.ShapeDtypeStruct((NPAD, D), jnp.float32),
  )(zp2[0], zp2[1], y2, dinv, b2.reshape(1, D))

  return out[:N]


# speedup vs baseline: 1.0292x; 1.0292x over previous
import functools

import jax
import jax.numpy as jnp
from jax import lax
from jax.experimental import pallas as pl
from jax.experimental.pallas import tpu as pltpu
from jax.experimental.pallas import tpu_sc as plsc

N = 10000
NPAD = 10240
D = 128
NC, NS = 2, 16
NW = NC * NS
EB = 64
DB = 128
RPS = NPAD // NS

_MESH = plsc.VectorSubcoreMesh(core_axis_name="c", subcore_axis_name="s")


def _make_deg_kernel(e_pad):
  nb = e_pad // (NW * DB)

  @functools.partial(
      pl.kernel,
      out_type=jax.ShapeDtypeStruct((NC, NPAD), jnp.float32),
      mesh=_MESH,
      scratch_types=[
          pltpu.VMEM((DB,), jnp.int32),
          pltpu.VMEM((DB,), jnp.float32),
          pltpu.VMEM((RPS,), jnp.float32),
          pltpu.VMEM_SHARED((NPAD,), jnp.float32),
      ],
  )
  def deg_kernel(dst_hbm, out_hbm, idx_v, ones_v, zbuf_v, acc):
    c = lax.axis_index("c")
    s = lax.axis_index("s")
    wid = s * NC + c

    def fill_ones(i, _):
      ones_v[pl.ds(i * 16, 16)] = jnp.ones((16,), jnp.float32)
      return 0

    lax.fori_loop(0, DB // 16, fill_ones, 0)

    def fill_zero(i, _):
      zbuf_v[pl.ds(i * 16, 16)] = jnp.zeros((16,), jnp.float32)
      return 0

    lax.fori_loop(0, RPS // 16, fill_zero, 0)
    pltpu.sync_copy(zbuf_v, acc.at[pl.ds(s * RPS, RPS)])
    plsc.subcore_barrier()

    base = wid * (nb * DB)

    def body(j, _):
      pltpu.sync_copy(dst_hbm.at[pl.ds(base + j * DB, DB)], idx_v)
      pltpu.sync_copy(ones_v, acc.at[idx_v], add=True)
      return 0

    lax.fori_loop(0, nb, body, 0)
    plsc.subcore_barrier()
    pltpu.sync_copy(acc.at[pl.ds(s * RPS, RPS)],
                    out_hbm.at[c, pl.ds(s * RPS, RPS)])

  return deg_kernel


CH = 16


def _make_scatter_kernel(e_pad):
  nb = e_pad // (NW * EB)
  assert nb % CH == 0

  @functools.partial(
      pl.kernel,
      out_type=jax.ShapeDtypeStruct((NC, NPAD, D), jnp.float32),
      mesh=_MESH,
      scratch_types=[
          pltpu.VMEM((CH, EB), jnp.int32),
          pltpu.VMEM((EB,), jnp.int32),
          pltpu.VMEM((EB,), jnp.int32),
          pltpu.VMEM((EB, D), jnp.float32),
          pltpu.VMEM((EB, D), jnp.float32),
          pltpu.VMEM((EB, D), jnp.float32),
          pltpu.VMEM((EB, D), jnp.float32),
          pltpu.VMEM_SHARED((NPAD, D), jnp.float32),
          [pltpu.SemaphoreType.DMA] * 4,
          [pltpu.SemaphoreType.DMA] * 4,
      ],
  )
  def scat_kernel(y_hbm, src_hbm, dst_hbm, out_hbm,
                  sidx_v, didx0_v, didx1_v, r0, r1, r2, r3, acc, gsem, ssem):
    c = lax.axis_index("c")
    s = lax.axis_index("s")
    wid = s * NC + c
    rows = (r0, r1, r2, r3)
    didx = (didx0_v, didx1_v)

    def zero_rows(i, _):
      r0[i // (D // 16), pl.ds((i % (D // 16)) * 16, 16)] = (
          jnp.zeros((16,), jnp.float32))
      return 0

    lax.fori_loop(0, EB * (D // 16), zero_rows, 0)

    def zero_acc(j, _):
      pltpu.sync_copy(r0, acc.at[pl.ds(s * RPS + j * EB, EB)])
      return 0

    lax.fori_loop(0, RPS // EB, zero_acc, 0)
    plsc.subcore_barrier()

    base = wid * (nb * EB)

    def chunk_body(ci, _):
      pltpu.sync_copy(src_hbm.at[wid, pl.ds(ci * CH, CH)], sidx_v)
      pltpu.async_copy(y_hbm.at[sidx_v.at[0]], rows[0], gsem[0])
      pltpu.async_copy(y_hbm.at[sidx_v.at[1]], rows[1], gsem[1])
      for k in range(CH):
        b = k % 4
        if k >= 2:
          pltpu.make_async_copy(rows[(k + 2) % 4], acc.at[didx[k % 2]],
                                ssem[(k + 2) % 4]).wait()
        if k + 2 < CH:
          pltpu.async_copy(y_hbm.at[sidx_v.at[k + 2]], rows[(k + 2) % 4],
                           gsem[(k + 2) % 4])
        pltpu.sync_copy(
            dst_hbm.at[pl.ds(base + (ci * CH + k) * EB, EB)], didx[k % 2])
        pltpu.make_async_copy(y_hbm.at[sidx_v.at[k]], rows[b], gsem[b]).wait()
        pltpu.async_copy(rows[b], acc.at[didx[k % 2]], ssem[b], add=True)
      for t in range(CH - 2, CH):
        pltpu.make_async_copy(rows[t % 4], acc.at[didx[t % 2]],
                              ssem[t % 4]).wait()
      return 0

    lax.fori_loop(0, nb // CH, chunk_body, 0)
    plsc.subcore_barrier()

    def writeback(j, _):
      pltpu.sync_copy(acc.at[pl.ds(s * RPS + j * EB, EB)],
                      out_hbm.at[c, pl.ds(s * RPS + j * EB, EB)])
      return 0

    lax.fori_loop(0, RPS // EB, writeback, 0)

  return scat_kernel


RB = 1024
_TC_GRID = NPAD // RB


def _lin1_body(x_ref, w_ref, d0_ref, d1_ref, y_ref, dinv_ref):
  deg = d0_ref[...] + d1_ref[...] + 1.0
  dinv = lax.rsqrt(deg)
  dinv_ref[...] = dinv
  xw = jnp.dot(x_ref[...], w_ref[...], preferred_element_type=jnp.float32)
  y_ref[...] = xw * dinv


def _lin2_body(z0_ref, z1_ref, y1_ref, dinv_ref, b1_ref, w_ref, y2_ref):
  dinv = dinv_ref[...]
  z = z0_ref[...] + z1_ref[...] + y1_ref[...]
  h = jnp.maximum(z * dinv + b1_ref[...], 0.0)
  hw = jnp.dot(h, w_ref[...], preferred_element_type=jnp.float32)
  y2_ref[...] = hw * dinv


def _out_body(z0_ref, z1_ref, y2_ref, dinv_ref, b2_ref, o_ref):
  z = z0_ref[...] + z1_ref[...] + y2_ref[...]
  o_ref[...] = z * dinv_ref[...] + b2_ref[...]


def _rows_spec():
  return pl.BlockSpec((RB, D), lambda i: (i, 0))


def _col_spec():
  return pl.BlockSpec((RB, 1), lambda i: (i, 0))


def _full_spec(r):
  return pl.BlockSpec((r, D), lambda i: (0, 0))


def kernel(x, edge_index, W1, b1, W2, b2):
  src = edge_index[0].astype(jnp.int32)
  dst = edge_index[1].astype(jnp.int32)
  e = src.shape[0]
  chunk = CH * EB * NW
  e_pad = chunk * (-(-e // chunk))
  nb = e_pad // (NW * EB)
  pad = N + (jnp.arange(e_pad - e, dtype=jnp.int32) % (NPAD - N))
  src_p = jnp.concatenate([src, pad])
  dst_p = jnp.concatenate([dst, pad])
  src3 = src_p.reshape(NW, nb, EB)
  x_p = jnp.pad(x, ((0, NPAD - N), (0, 0)))

  deg_fn = _make_deg_kernel(e_pad)
  scat_fn = _make_scatter_kernel(e_pad)

  degp = deg_fn(dst_p)
  d0 = degp[0][:, None]
  d1 = degp[1][:, None]

  y1, dinv = pl.pallas_call(
      _lin1_body,
      grid=(_TC_GRID,),
      in_specs=[_rows_spec(), _full_spec(D), _col_spec(), _col_spec()],
      out_specs=[_rows_spec(), _col_spec()],
      out_shape=[
          jax.ShapeDtypeStruct((NPAD, D), jnp.float32),
          jax.ShapeDtypeStruct((NPAD, 1), jnp.float32),
      ],
  )(x_p, W1, d0, d1)

  zp1 = scat_fn(y1, src3, dst_p)

  y2 = pl.pallas_call(
      _lin2_body,
      grid=(_TC_GRID,),
      in_specs=[_rows_spec(), _rows_spec(), _rows_spec(), _col_spec(),
                _full_spec(1), _full_spec(D)],
      out_specs=_rows_spec(),
      out_shape=jax.ShapeDtypeStruct((NPAD, D), jnp.float32),
  )(zp1[0], zp1[1], y1, dinv, b1.reshape(1, D), W2)

  zp2 = scat_fn(y2, src3, dst_p)

  out = pl.pallas_call(
      _out_body,
      grid=(_TC_GRID,),
      in_specs=[_rows_spec(), _rows_spec(), _rows_spec(), _col_spec(),
                _full_spec(1)],
      out_specs=_rows_spec(),
      out_shape=jax.ShapeDtypeStruct((NPAD, D), jnp.float32),
  )(zp2[0], zp2[1], y2, dinv, b2.reshape(1, D))

  return out[:N]
